# baseline (speedup 1.0000x reference)
import functools

import jax
import jax.numpy as jnp
from jax import lax
from jax.experimental import pallas as pl
from jax.experimental.pallas import tpu as pltpu
from jax.experimental.pallas import tpu_sc as plsc

N = 1_000_000
NPROJ = 8
MIN_RATIO = 1500.0 / 30000000.0
L = 16
HBINS = 65536
TPP = 4

_XBASE = 256
_VECW = 2 * L
_VSLOT = 16 * _VECW
_SBASE = _XBASE + 2 * _VSLOT
_SLICEW = 2 * 256
_SSLOT = 16 * _SLICEW
_SHR_WORDS = _SBASE + 2 * _SSLOT


def _iota():
    return lax.broadcasted_iota(jnp.int32, (L,), 0)


def _lane(v, g):
    return jnp.sum(jnp.where(_iota() == g, v, 0))


def _zero_hist(h):
    z = jnp.zeros((L,), jnp.int32)

    @plsc.parallel_loop(0, HBINS // L, unroll=8)
    def _(i):
        h[pl.ds(i * L, L)] = z


def _child_totals(h, base, span):
    child = span // 16
    if child == 1:
        return h[pl.ds(base, L)]
    nv = child // L
    T = jnp.zeros((L,), jnp.int32)
    for g in range(16):
        start = base + g * child
        if nv <= 4:
            acc = jnp.zeros((L,), jnp.int32)
            for j in range(nv):
                acc = acc + h[pl.ds(start + j * L, L)]
        else:
            @plsc.parallel_loop(0, nv, unroll=8,
                                carry=jnp.zeros((L,), jnp.int32))
            def acc(j, a, start=start):
                return a + h[pl.ds(start + j * L, L)]
        T = jnp.where(_iota() == g, jnp.sum(acc), T)
    return T


def _step(T, r, cbelow):
    inc = plsc.cumsum(T)
    g = jnp.max(plsc.all_reduce_ffs((cbelow + inc) > r))
    return g, cbelow + _lane(inc, g) - _lane(T, g)


def _build_sc_median(n, chunk, unroll, interpret=False):
    per_tile = n // TPP
    nchunk = per_tile // chunk
    vpc = chunk // L
    assert nchunk * chunk == per_tile and vpc * L == chunk
    assert nchunk % 2 == 1 and vpc % unroll == 0
    npair = (nchunk - 1) // 2
    r0 = n - (n // 2 + 1)
    r1 = n // 2

    mesh = plsc.VectorSubcoreMesh(
        core_axis_name="c", subcore_axis_name="s", num_cores=2, num_subcores=16
    )

    @functools.partial(
        pl.kernel,
        out_type=jax.ShapeDtypeStruct((NPROJ, L), jnp.float32),
        mesh=mesh,
        interpret=interpret,
        compiler_params=pltpu.CompilerParams(needs_layout_passes=False),
        scratch_types=[
            pltpu.VMEM((chunk,), jnp.float32),
            pltpu.VMEM((chunk,), jnp.float32),
            pltpu.VMEM((HBINS,), jnp.int32),
            pltpu.VMEM((L,), jnp.float32),
            pltpu.VMEM((_VECW,), jnp.int32),
            pltpu.VMEM((TPP * _VECW,), jnp.int32),
            pltpu.VMEM((TPP * _SLICEW,), jnp.int32),
            pltpu.VMEM_SHARED((_SHR_WORDS,), jnp.int32),
            pltpu.SemaphoreType.DMA,
            pltpu.SemaphoreType.DMA,
        ],
    )
    def sc_median(x0, x1, x2, x3, x4, x5, x6, x7, out, buf0, buf1, hist,
                  med_v, xbuf, tbuf, tslice, shr, sem0, sem1):
        xs = [x0, x1, x2, x3, x4, x5, x6, x7]
        c = lax.axis_index("c")
        s = lax.axis_index("s")
        p_l = lax.div(s, TPP)
        q = lax.rem(s, TPP)
        p = c * 4 + p_l
        ones = jnp.ones((L,), jnp.int32)
        rnd = [0]

        def exchange2(va, vb, comb_a, comb_b):
            base = _XBASE + (rnd[0] % 2) * _VSLOT
            rnd[0] += 1
            xbuf[pl.ds(0, L)] = va
            xbuf[pl.ds(L, L)] = vb
            pltpu.sync_copy(xbuf, shr.at[pl.ds(base + s * _VECW, _VECW)])
            plsc.subcore_barrier()
            pltpu.sync_copy(
                shr.at[pl.ds(base + (p_l * TPP) * _VECW, TPP * _VECW)], tbuf
            )
            acc_a = None
            acc_b = None
            for r in range(TPP):
                ta = tbuf[pl.ds(r * _VECW, L)]
                tb = tbuf[pl.ds(r * _VECW + L, L)]
                acc_a = ta if acc_a is None else comb_a(acc_a, ta)
                acc_b = tb if acc_b is None else comb_b(acc_b, tb)
            return acc_a, acc_b

        def exchange_slices(ba, bb):
            base = _SBASE + (rnd[0] % 2) * _SSLOT
            rnd[0] += 1
            dst = base + s * _SLICEW
            pltpu.sync_copy(hist.at[pl.ds(pl.multiple_of(ba, 8), 256)],
                            shr.at[pl.ds(dst, 256)])
            pltpu.sync_copy(hist.at[pl.ds(pl.multiple_of(bb, 8), 256)],
                            shr.at[pl.ds(dst + 256, 256)])
            plsc.subcore_barrier()
            pltpu.sync_copy(
                shr.at[pl.ds(base + (p_l * TPP) * _SLICEW, TPP * _SLICEW)],
                tslice,
            )

        def _slice_vreg(off, i):
            acc = tslice[pl.ds(off + i * L, L)]
            for r in range(1, TPP):
                acc = acc + tslice[pl.ds(r * _SLICEW + off + i * L, L)]
            return acc

        def _finish256(off, r, cbelow):
            T = jnp.zeros((L,), jnp.int32)
            for g in range(16):
                T = jnp.where(_iota() == g, jnp.sum(_slice_vreg(off, g)), T)
            g1, cbelow = _step(T, r, cbelow)
            V = _slice_vreg(off, g1)
            g2, cbelow = _step(V, r, cbelow)
            return g1 * 16 + g2, cbelow

        def find2(ra, rb, T0):
            gA, ca = _step(T0, ra, jnp.int32(0))
            gB, cb = _step(T0, rb, jnp.int32(0))
            ba = gA * 4096
            bb = gB * 4096
            TA, TB = exchange2(
                _child_totals(hist, ba, 4096), _child_totals(hist, bb, 4096),
                jnp.add, jnp.add,
            )
            gA, ca = _step(TA, ra, ca)
            gB, cb = _step(TB, rb, cb)
            ba = ba + gA * 256
            bb = bb + gB * 256
            exchange_slices(ba, bb)
            la, ca = _finish256(0, ra, ca)
            lb, cb = _finish256(256, rb, cb)
            return ba + la, ca, bb + lb, cb

        def start_load(ci, buf, sem):
            off = pl.multiple_of(q * per_tile + ci * chunk, 8)
            for i in range(NPROJ):
                @pl.when(p == i)
                def _(i=i):
                    pltpu.make_async_copy(
                        xs[i].at[pl.ds(off, chunk)], buf, sem
                    ).start()

        def wait_load(buf, sem):
            pltpu.make_async_copy(x0.at[pl.ds(0, chunk)], buf, sem).wait()

        def stream(process, carry, skip_first_start=False):
            if not skip_first_start:
                start_load(0, buf0, sem0)

            def pair(g, carry):
                a = 2 * g
                wait_load(buf0, sem0)
                start_load(a + 1, buf1, sem1)
                carry = process(buf0, carry)
                wait_load(buf1, sem1)
                start_load(a + 2, buf0, sem0)
                return process(buf1, carry)

            carry = lax.fori_loop(0, npair, pair, carry)
            wait_load(buf0, sem0)
            return process(buf0, carry)

        with jax.named_scope("zero1"):
            _zero_hist(hist)

        def proc1(buf, carry):
            @plsc.parallel_loop(0, vpc, unroll=unroll)
            def _(j):
                v = buf[pl.ds(j * L, L)]
                bits = lax.bitcast_convert_type(v, jnp.int32)
                hi = lax.shift_right_logical(bits, 16)
                plsc.addupdate_scatter(hist, [hi], ones)

            return carry

        with jax.named_scope("stream1"):
            stream(proc1, jnp.int32(0))
        start_load(0, buf0, sem0)

        with jax.named_scope("scan1"):
            myT0 = _child_totals(hist, 0, HBINS)
        with jax.named_scope("search1"):
            T0, _ = exchange2(myT0, ones, jnp.add, jnp.add)
            b_a, c_a, b_b, _ = find2(jnp.int32(r0), jnp.int32(r1), T0)

        with jax.named_scope("zero2"):
            _zero_hist(hist)

        def proc2(buf, carry):
            @plsc.parallel_loop(0, vpc, unroll=unroll)
            def _(j):
                v = buf[pl.ds(j * L, L)]
                bits = lax.bitcast_convert_type(v, jnp.int32)
                hi = lax.shift_right_logical(bits, 16)
                lo = lax.bitwise_and(bits, 0xFFFF)
                plsc.addupdate_scatter(hist, [lo], ones, mask=hi == b_a)

            return carry

        with jax.named_scope("stream2"):
            stream(proc2, jnp.int32(0), skip_first_start=True)

        xbuf[pl.ds(0, L)] = jnp.full((L,), 0x10000, jnp.int32)

        @pl.when(b_a != b_b)
        def _():
            def proc3(buf, rm):
                @plsc.parallel_loop(0, vpc, unroll=unroll, carry=rm)
                def rm(j, acc):
                    v = buf[pl.ds(j * L, L)]
                    bits = lax.bitcast_convert_type(v, jnp.int32)
                    hi = lax.shift_right_logical(bits, 16)
                    lo = lax.bitwise_and(bits, 0xFFFF)
                    return jnp.minimum(
                        acc, jnp.where(hi == b_b, lo, jnp.int32(0x10000))
                    )

                return rm

            xbuf[pl.ds(0, L)] = stream(
                proc3, jnp.full((L,), 0x10000, jnp.int32)
            )

        runmin = xbuf[pl.ds(0, L)]
        with jax.named_scope("scan2"):
            myT02 = _child_totals(hist, 0, HBINS)
        T02, runmin = exchange2(
            myT02, runmin, jnp.add, jnp.minimum
        )
        n_a = jnp.sum(T02)
        r1p = jnp.minimum(jnp.int32(r1) - c_a, n_a - 1)
        lo_a, _, lo_b, _ = find2(jnp.int32(r0) - c_a, r1p, T02)
        minlow = jnp.min(runmin)

        same = b_a == b_b
        v0_bits = lax.shift_left(b_a, 16) | lo_a
        v1_bits = jnp.where(
            same,
            lax.shift_left(b_a, 16) | lo_b,
            lax.shift_left(b_b, 16) | minlow,
        )
        f0 = lax.bitcast_convert_type(jnp.full((L,), v0_bits), jnp.float32)
        f1 = lax.bitcast_convert_type(jnp.full((L,), v1_bits), jnp.float32)
        med_v[...] = (f0 + f1) * 0.5

        @pl.when(q == 0)
        def _():
            pltpu.sync_copy(med_v, out.at[p])

    return sc_median


_sc_median = _build_sc_median(N, 10000, 5)


def _combine_body(m_ref, o_ref):
    med = m_ref[...][:, 0:1]
    total = jnp.sum(med)
    ratio = med / total
    meets = (ratio >= jnp.float32(MIN_RATIO)).astype(jnp.float32)
    o_ref[...] = 30000000 * ratio * meets


@jax.jit
def kernel(x0, x1, x2, x3, x4, x5, x6, x7):
    meds = _sc_median(x0, x1, x2, x3, x4, x5, x6, x7)
    return pl.pallas_call(
        _combine_body,
        out_shape=jax.ShapeDtypeStruct((NPROJ, 1), jnp.float32),
    )(meds)


# speedup vs baseline: 1.0144x; 1.0144x over previous
import functools

import jax
import jax.numpy as jnp
from jax import lax
from jax.experimental import pallas as pl
from jax.experimental.pallas import tpu as pltpu
from jax.experimental.pallas import tpu_sc as plsc

N = 1_000_000
NPROJ = 8
MIN_RATIO = 1500.0 / 30000000.0
L = 16
HBINS = 65536
TPP = 4

_XBASE = 256
_VECW = 2 * L
_VSLOT = 16 * _VECW
_SBASE = _XBASE + 2 * _VSLOT
_SLICEW = 2 * 256
_SSLOT = 16 * _SLICEW
_SHR_WORDS = _SBASE + 2 * _SSLOT


def _iota():
    return lax.broadcasted_iota(jnp.int32, (L,), 0)


def _lane(v, g):
    return jnp.sum(jnp.where(_iota() == g, v, 0))


def _zero_hist(h):
    z = jnp.zeros((L,), jnp.int32)

    @plsc.parallel_loop(0, HBINS // L, unroll=8)
    def _(i):
        h[pl.ds(i * L, L)] = z


def _child_totals(h, base, span):
    child = span // 16
    if child == 1:
        return h[pl.ds(base, L)]
    nv = child // L
    T = jnp.zeros((L,), jnp.int32)
    for g in range(16):
        start = base + g * child
        if nv <= 4:
            acc = jnp.zeros((L,), jnp.int32)
            for j in range(nv):
                acc = acc + h[pl.ds(start + j * L, L)]
        else:
            @plsc.parallel_loop(0, nv, unroll=8,
                                carry=jnp.zeros((L,), jnp.int32))
            def acc(j, a, start=start):
                return a + h[pl.ds(start + j * L, L)]
        T = jnp.where(_iota() == g, jnp.sum(acc), T)
    return T


def _step(T, r, cbelow):
    inc = plsc.cumsum(T)
    g = jnp.max(plsc.all_reduce_ffs((cbelow + inc) > r))
    return g, cbelow + _lane(inc, g) - _lane(T, g)


def _build_sc_median(n, chunk, unroll, interpret=False):
    per_tile = n // TPP
    nchunk = per_tile // chunk
    vpc = chunk // L
    assert nchunk * chunk == per_tile and vpc * L == chunk
    assert nchunk % 2 == 1 and vpc % unroll == 0
    npair = (nchunk - 1) // 2
    r0 = n - (n // 2 + 1)
    r1 = n // 2

    mesh = plsc.VectorSubcoreMesh(
        core_axis_name="c", subcore_axis_name="s", num_cores=2, num_subcores=16
    )

    @functools.partial(
        pl.kernel,
        out_type=jax.ShapeDtypeStruct((NPROJ, L), jnp.float32),
        mesh=mesh,
        interpret=interpret,
        compiler_params=pltpu.CompilerParams(needs_layout_passes=False),
        scratch_types=[
            pltpu.VMEM((chunk,), jnp.float32),
            pltpu.VMEM((chunk,), jnp.float32),
            pltpu.VMEM((HBINS,), jnp.int32),
            pltpu.VMEM((L,), jnp.float32),
            pltpu.VMEM((_VECW,), jnp.int32),
            pltpu.VMEM((TPP * _VECW,), jnp.int32),
            pltpu.VMEM((TPP * _SLICEW,), jnp.int32),
            pltpu.VMEM_SHARED((_SHR_WORDS,), jnp.int32),
            pltpu.SemaphoreType.DMA,
            pltpu.SemaphoreType.DMA,
        ],
    )
    def sc_median(x0, x1, x2, x3, x4, x5, x6, x7, out, buf0, buf1, hist,
                  med_v, xbuf, tbuf, tslice, shr, sem0, sem1):
        xs = [x0, x1, x2, x3, x4, x5, x6, x7]
        c = lax.axis_index("c")
        s = lax.axis_index("s")
        p_l = lax.div(s, TPP)
        q = lax.rem(s, TPP)
        p = c * 4 + p_l
        ones = jnp.ones((L,), jnp.int32)
        rnd = [0]

        def exchange2(va, vb, comb_a, comb_b):
            base = _XBASE + (rnd[0] % 2) * _VSLOT
            rnd[0] += 1
            xbuf[pl.ds(0, L)] = va
            xbuf[pl.ds(L, L)] = vb
            pltpu.sync_copy(xbuf, shr.at[pl.ds(base + s * _VECW, _VECW)])
            plsc.subcore_barrier()
            pltpu.sync_copy(
                shr.at[pl.ds(base + (p_l * TPP) * _VECW, TPP * _VECW)], tbuf
            )
            acc_a = None
            acc_b = None
            for r in range(TPP):
                ta = tbuf[pl.ds(r * _VECW, L)]
                tb = tbuf[pl.ds(r * _VECW + L, L)]
                acc_a = ta if acc_a is None else comb_a(acc_a, ta)
                acc_b = tb if acc_b is None else comb_b(acc_b, tb)
            return acc_a, acc_b

        def exchange_slices(ba, bb):
            base = _SBASE + (rnd[0] % 2) * _SSLOT
            rnd[0] += 1
            dst = base + s * _SLICEW
            pltpu.sync_copy(hist.at[pl.ds(pl.multiple_of(ba, 8), 256)],
                            shr.at[pl.ds(dst, 256)])
            pltpu.sync_copy(hist.at[pl.ds(pl.multiple_of(bb, 8), 256)],
                            shr.at[pl.ds(dst + 256, 256)])
            plsc.subcore_barrier()
            pltpu.sync_copy(
                shr.at[pl.ds(base + (p_l * TPP) * _SLICEW, TPP * _SLICEW)],
                tslice,
            )

        def _slice_vreg(off, i):
            acc = tslice[pl.ds(off + i * L, L)]
            for r in range(1, TPP):
                acc = acc + tslice[pl.ds(r * _SLICEW + off + i * L, L)]
            return acc

        def _finish256(off, r, cbelow):
            T = jnp.zeros((L,), jnp.int32)
            for g in range(16):
                T = jnp.where(_iota() == g, jnp.sum(_slice_vreg(off, g)), T)
            g1, cbelow = _step(T, r, cbelow)
            V = _slice_vreg(off, g1)
            g2, cbelow = _step(V, r, cbelow)
            return g1 * 16 + g2, cbelow

        def find2(ra, rb, T0):
            gA, ca = _step(T0, ra, jnp.int32(0))
            gB, cb = _step(T0, rb, jnp.int32(0))
            ba = gA * 4096
            bb = gB * 4096
            TA, TB = exchange2(
                _child_totals(hist, ba, 4096), _child_totals(hist, bb, 4096),
                jnp.add, jnp.add,
            )
            gA, ca = _step(TA, ra, ca)
            gB, cb = _step(TB, rb, cb)
            ba = ba + gA * 256
            bb = bb + gB * 256
            exchange_slices(ba, bb)
            la, ca = _finish256(0, ra, ca)
            lb, cb = _finish256(256, rb, cb)
            return ba + la, ca, bb + lb, cb

        def start_load(ci, buf, sem):
            off = pl.multiple_of(q * per_tile + ci * chunk, 8)
            for i in range(NPROJ):
                @pl.when(p == i)
                def _(i=i):
                    pltpu.make_async_copy(
                        xs[i].at[pl.ds(off, chunk)], buf, sem
                    ).start()

        def wait_load(buf, sem):
            pltpu.make_async_copy(x0.at[pl.ds(0, chunk)], buf, sem).wait()

        def stream(process, carry, skip_first_start=False):
            if not skip_first_start:
                start_load(0, buf0, sem0)

            def pair(g, carry):
                a = 2 * g
                wait_load(buf0, sem0)
                start_load(a + 1, buf1, sem1)
                carry = process(buf0, carry)
                wait_load(buf1, sem1)
                start_load(a + 2, buf0, sem0)
                return process(buf1, carry)

            carry = lax.fori_loop(0, npair, pair, carry)
            wait_load(buf0, sem0)
            return process(buf0, carry)

        start_load(0, buf0, sem0)
        _zero_hist(hist)

        def proc1(buf, carry):
            @plsc.parallel_loop(0, vpc, unroll=unroll)
            def _(j):
                v = buf[pl.ds(j * L, L)]
                bits = lax.bitcast_convert_type(v, jnp.int32)
                hi = lax.shift_right_logical(bits, 16)
                plsc.addupdate_scatter(hist, [hi], ones)

            return carry

        stream(proc1, jnp.int32(0), skip_first_start=True)
        start_load(0, buf0, sem0)

        T0, _ = exchange2(_child_totals(hist, 0, HBINS), ones, jnp.add,
                          jnp.add)
        b_a, c_a, b_b, _ = find2(jnp.int32(r0), jnp.int32(r1), T0)

        _zero_hist(hist)

        def proc2(buf, carry):
            @plsc.parallel_loop(0, vpc, unroll=unroll)
            def _(j):
                v = buf[pl.ds(j * L, L)]
                bits = lax.bitcast_convert_type(v, jnp.int32)
                hi = lax.shift_right_logical(bits, 16)
                lo = lax.bitwise_and(bits, 0xFFFF)
                plsc.addupdate_scatter(hist, [lo], ones, mask=hi == b_a)

            return carry

        stream(proc2, jnp.int32(0), skip_first_start=True)

        xbuf[pl.ds(0, L)] = jnp.full((L,), 0x10000, jnp.int32)

        @pl.when(b_a != b_b)
        def _():
            def proc3(buf, rm):
                @plsc.parallel_loop(0, vpc, unroll=unroll, carry=rm)
                def rm(j, acc):
                    v = buf[pl.ds(j * L, L)]
                    bits = lax.bitcast_convert_type(v, jnp.int32)
                    hi = lax.shift_right_logical(bits, 16)
                    lo = lax.bitwise_and(bits, 0xFFFF)
                    return jnp.minimum(
                        acc, jnp.where(hi == b_b, lo, jnp.int32(0x10000))
                    )

                return rm

            xbuf[pl.ds(0, L)] = stream(
                proc3, jnp.full((L,), 0x10000, jnp.int32)
            )

        runmin = xbuf[pl.ds(0, L)]
        T02, runmin = exchange2(
            _child_totals(hist, 0, HBINS), runmin, jnp.add, jnp.minimum
        )
        n_a = jnp.sum(T02)
        r1p = jnp.minimum(jnp.int32(r1) - c_a, n_a - 1)
        lo_a, _, lo_b, _ = find2(jnp.int32(r0) - c_a, r1p, T02)
        minlow = jnp.min(runmin)

        same = b_a == b_b
        v0_bits = lax.shift_left(b_a, 16) | lo_a
        v1_bits = jnp.where(
            same,
            lax.shift_left(b_a, 16) | lo_b,
            lax.shift_left(b_b, 16) | minlow,
        )
        f0 = lax.bitcast_convert_type(jnp.full((L,), v0_bits), jnp.float32)
        f1 = lax.bitcast_convert_type(jnp.full((L,), v1_bits), jnp.float32)
        med_v[...] = (f0 + f1) * 0.5

        @pl.when(q == 0)
        def _():
            pltpu.sync_copy(med_v, out.at[p])

    return sc_median


_sc_median = _build_sc_median(N, 10000, 5)


def _combine_body(m_ref, o_ref):
    med = m_ref[...][:, 0:1]
    total = jnp.sum(med)
    ratio = med / total
    meets = (ratio >= jnp.float32(MIN_RATIO)).astype(jnp.float32)
    o_ref[...] = 30000000 * ratio * meets


@jax.jit
def kernel(x0, x1, x2, x3, x4, x5, x6, x7):
    meds = _sc_median(x0, x1, x2, x3, x4, x5, x6, x7)
    return pl.pallas_call(
        _combine_body,
        out_shape=jax.ShapeDtypeStruct((NPROJ, 1), jnp.float32),
    )(meds)
